# baseline (speedup 1.0000x reference)
import jax
import jax.numpy as jnp
from jax.experimental import pallas as pl
from jax.experimental.pallas import tpu as pltpu

N = 8192
K = 8192
D = 32
BN = 512
BC = BN // 128


def _bits(x):
    return jax.lax.bitcast_convert_type(x, jnp.int32)


def _float(b):
    return jax.lax.bitcast_convert_type(b, jnp.float32)


def _to_key(b):
    return jnp.where(b < 0, b ^ jnp.int32(0x7FFFFFFF), b)


def _vq_kernel(z_ref, w_ref, logits_ref, idx_ref):
    logits = jax.lax.dot_general(
        z_ref[...], w_ref[...],
        dimension_numbers=(((1,), (1,)), ((), ())),
        preferred_element_type=jnp.float32,
    )
    logits_ref[...] = logits

    m = jnp.max(logits, axis=1, keepdims=True)
    e = jnp.exp(logits - m)
    s = jnp.sum(e, axis=1, keepdims=True)

    m_c = m.reshape(BC, 128)
    s_c = s.reshape(BC, 128)
    ymax_c = jnp.exp(m_c - m_c) / s_c

    def y_of(x):
        return jnp.exp(x - m_c) / s_c

    lo = _to_key(_bits(m_c - jnp.float32(32.0)))
    hi = _to_key(_bits(m_c))
    for _ in range(32):
        mid = (lo >> 1) + (hi >> 1) + (lo & hi & 1)
        x = _float(jnp.where(mid < 0, mid ^ jnp.int32(0x7FFFFFFF), mid))
        ok = y_of(x) >= ymax_c
        hi = jnp.where(ok, mid, hi)
        lo = jnp.where(ok, lo, mid + 1)
    t_key = jnp.where(hi < 0, hi ^ jnp.int32(0x7FFFFFFF), hi)
    thresh = _float(t_key).reshape(BN, 1)

    iota = jax.lax.broadcasted_iota(jnp.int32, (1, K), 1).astype(jnp.float32)
    cand = jnp.where(logits >= thresh, iota, jnp.float32(K))
    idx_ref[0, 0, :] = jnp.min(cand, axis=1).astype(jnp.int32)


def kernel(z_e_x, W):
    grid = (N // BN,)
    logits, idx = pl.pallas_call(
        _vq_kernel,
        grid=grid,
        in_specs=[
            pl.BlockSpec((BN, D), lambda i: (i, 0)),
            pl.BlockSpec((K, D), lambda i: (0, 0)),
        ],
        out_specs=[
            pl.BlockSpec((BN, K), lambda i: (i, 0)),
            pl.BlockSpec((1, 1, BN), lambda i: (i, 0, 0)),
        ],
        out_shape=[
            jax.ShapeDtypeStruct((N, K), jnp.float32),
            jax.ShapeDtypeStruct((N // BN, 1, BN), jnp.int32),
        ],
        compiler_params=pltpu.CompilerParams(
            dimension_semantics=("parallel",),
        ),
    )(z_e_x, W)
    return (logits, idx.reshape(N))


# speedup vs baseline: 1.6040x; 1.6040x over previous
import jax
import jax.numpy as jnp
from jax.experimental import pallas as pl
from jax.experimental.pallas import tpu as pltpu

N = 8192
K = 8192
D = 32
BN = 512


def _vq_kernel(z_ref, w_ref, logits_ref, idx_ref):
    logits = jax.lax.dot_general(
        z_ref[...], w_ref[...],
        dimension_numbers=(((1,), (1,)), ((), ())),
        preferred_element_type=jnp.float32,
    )
    logits_ref[...] = logits
    m = jnp.max(logits, axis=1, keepdims=True)
    e = jnp.exp(logits - m)
    s = jnp.sum(e, axis=1, keepdims=True)
    emax = jnp.exp(m - m)
    ymax = emax / s
    iota = jax.lax.broadcasted_iota(jnp.int32, (1, K), 1).astype(jnp.float32)
    cand = jnp.where(e / s == ymax, iota, jnp.float32(K))
    idx_ref[0, 0, :] = jnp.min(cand, axis=1).astype(jnp.int32)


def kernel(z_e_x, W):
    grid = (N // BN,)
    logits, idx = pl.pallas_call(
        _vq_kernel,
        grid=grid,
        in_specs=[
            pl.BlockSpec((BN, D), lambda i: (i, 0)),
            pl.BlockSpec((K, D), lambda i: (0, 0)),
        ],
        out_specs=[
            pl.BlockSpec((BN, K), lambda i: (i, 0)),
            pl.BlockSpec((1, 1, BN), lambda i: (i, 0, 0)),
        ],
        out_shape=[
            jax.ShapeDtypeStruct((N, K), jnp.float32),
            jax.ShapeDtypeStruct((N // BN, 1, BN), jnp.int32),
        ],
        compiler_params=pltpu.CompilerParams(
            dimension_semantics=("parallel",),
        ),
    )(z_e_x, W)
    return (logits, idx.reshape(N))


# speedup vs baseline: 1.6354x; 1.0195x over previous
import jax
import jax.numpy as jnp
from jax.experimental import pallas as pl
from jax.experimental.pallas import tpu as pltpu

N = 8192
K = 8192
D = 32
BN = 512
BC = BN // 128


def _vq_kernel(z_ref, w_ref, logits_ref, idx_ref):
    logits = jax.lax.dot_general(
        z_ref[...], w_ref[...],
        dimension_numbers=(((1,), (1,)), ((), ())),
        preferred_element_type=jnp.float32,
    )
    logits_ref[...] = logits
    lv = logits_ref[...]
    m = jnp.max(lv, axis=1, keepdims=True)
    e = jnp.exp(lv - m)
    s = jnp.sum(e, axis=1, keepdims=True)
    m_c = jnp.transpose(m, (1, 0))
    s_c = jnp.transpose(s, (1, 0))
    emax_c = jnp.exp(m_c - m_c)
    ymax_c = emax_c / s_c

    x0 = ymax_c * s_c
    k0 = jax.lax.bitcast_convert_type(x0, jnp.int32)
    lo = k0 - jnp.int32(2048)
    hi = k0 + jnp.int32(2048)
    for _ in range(12):
        mid = (lo >> 1) + (hi >> 1) + (lo & hi & 1)
        xm = jax.lax.bitcast_convert_type(mid, jnp.float32)
        ok = (xm / s_c) >= ymax_c
        hi = jnp.where(ok, mid, hi)
        lo = jnp.where(ok, lo, mid + 1)
    thresh = jnp.transpose(
        jax.lax.bitcast_convert_type(hi, jnp.float32), (1, 0))


    iota = jax.lax.broadcasted_iota(jnp.int32, (1, K), 1).astype(jnp.float32)
    cand = jnp.where(e >= thresh, iota, jnp.float32(K))
    idx_ref[0, 0, :] = jnp.min(cand, axis=1).astype(jnp.int32)


def kernel(z_e_x, W):
    grid = (N // BN,)
    logits, idx = pl.pallas_call(
        _vq_kernel,
        grid=grid,
        in_specs=[
            pl.BlockSpec((BN, D), lambda i: (i, 0)),
            pl.BlockSpec((K, D), lambda i: (0, 0)),
        ],
        out_specs=[
            pl.BlockSpec((BN, K), lambda i: (i, 0)),
            pl.BlockSpec((1, 1, BN), lambda i: (i, 0, 0)),
        ],
        out_shape=[
            jax.ShapeDtypeStruct((N, K), jnp.float32),
            jax.ShapeDtypeStruct((N // BN, 1, BN), jnp.int32),
        ],
        compiler_params=pltpu.CompilerParams(
            dimension_semantics=("parallel",),
        ),
    )(z_e_x, W)
    return (logits, idx.reshape(N))
